# baseline (speedup 1.0000x reference)
import math

import jax
import jax.numpy as jnp
import numpy as np
from jax.experimental import pallas as pl
from jax.experimental.pallas import tpu as pltpu

_LOG_NC = math.log(256.0)
_NEG = float(np.log(np.float32(1e-30)))


def _qsample_kernel(t_ref, lca_ref, l1m_ref, lx_ref, u_ref, out_ref):
    b = pl.program_id(0)
    ti = t_ref[b]
    a = lca_ref[ti]
    c = l1m_ref[ti] - _LOG_NC

    lx = lx_ref[0]
    u = u_ref[0]
    gumbel = -jnp.log(-jnp.log(u + 1e-30) + 1e-30)
    xa = lx + a
    m = jnp.maximum(xa, c)
    log_probs = m + jnp.log(jnp.exp(xa - m) + jnp.exp(c - m))
    v = gumbel + log_probs

    idx = jnp.argmax(v, axis=0)
    cls = jax.lax.broadcasted_iota(jnp.int32, v.shape, 0)
    out_ref[0] = jnp.where(cls == idx[None, :], jnp.float32(0.0),
                           jnp.float32(_NEG))


def kernel(log_x_start, t, uniform, log_cumprod_alpha, log_1_min_cumprod_alpha):
    B, C, H, W = log_x_start.shape
    HW = H * W
    L = 1024
    lx = log_x_start.reshape(B, C, HW)
    u = uniform.reshape(B, C, HW)
    grid = (B, HW // L)
    blk = pl.BlockSpec((1, C, L), lambda b, j: (b, 0, j))
    out = pl.pallas_call(
        _qsample_kernel,
        grid=grid,
        in_specs=[
            pl.BlockSpec(memory_space=pltpu.SMEM),
            pl.BlockSpec(memory_space=pltpu.SMEM),
            pl.BlockSpec(memory_space=pltpu.SMEM),
            blk,
            blk,
        ],
        out_specs=blk,
        out_shape=jax.ShapeDtypeStruct((B, C, HW), jnp.float32),
        compiler_params=pltpu.CompilerParams(
            dimension_semantics=("parallel", "parallel")),
    )(t, log_cumprod_alpha, log_1_min_cumprod_alpha, lx, u)
    return out.reshape(B, C, H, W)


# speedup vs baseline: 1.1056x; 1.1056x over previous
import math

import jax
import jax.numpy as jnp
import numpy as np
from jax.experimental import pallas as pl
from jax.experimental.pallas import tpu as pltpu

_LOG_NC = math.log(256.0)
_NEG = float(np.log(np.float32(1e-30)))


def _qsample_kernel(t_ref, lca_ref, l1m_ref, lx_ref, u_ref, out_ref):
    b = pl.program_id(0)
    ti = t_ref[b]
    a = lca_ref[ti]
    c = l1m_ref[ti] - _LOG_NC

    lx = lx_ref[0]
    u = u_ref[0]
    gumbel = -jnp.log(-jnp.log(u + 1e-30) + 1e-30)
    xa = lx + a
    m = jnp.maximum(xa, c)
    log_probs = m + jnp.log(jnp.exp(xa - m) + jnp.exp(c - m))
    v = gumbel + log_probs

    idx = jnp.argmax(v, axis=0)
    cls = jax.lax.broadcasted_iota(jnp.int32, v.shape, 0)
    out_ref[0] = jnp.where(cls == idx[None, :], jnp.float32(0.0),
                           jnp.float32(_NEG))


def kernel(log_x_start, t, uniform, log_cumprod_alpha, log_1_min_cumprod_alpha):
    B, C, H, W = log_x_start.shape
    HW = H * W
    L = 4096
    lx = log_x_start.reshape(B, C, HW)
    u = uniform.reshape(B, C, HW)
    grid = (B, HW // L)
    blk = pl.BlockSpec((1, C, L), lambda b, j: (b, 0, j))
    out = pl.pallas_call(
        _qsample_kernel,
        grid=grid,
        in_specs=[
            pl.BlockSpec(memory_space=pltpu.SMEM),
            pl.BlockSpec(memory_space=pltpu.SMEM),
            pl.BlockSpec(memory_space=pltpu.SMEM),
            blk,
            blk,
        ],
        out_specs=blk,
        out_shape=jax.ShapeDtypeStruct((B, C, HW), jnp.float32),
        compiler_params=pltpu.CompilerParams(
            dimension_semantics=("parallel", "parallel")),
    )(t, log_cumprod_alpha, log_1_min_cumprod_alpha, lx, u)
    return out.reshape(B, C, H, W)
